# baseline (speedup 1.0000x reference)
import functools

import jax
import jax.numpy as jnp
from jax import lax
from jax.experimental import pallas as pl
from jax.experimental.pallas import tpu as pltpu
from jax.experimental.pallas import tpu_sc as plsc

N = 10000
D = 128
H = 128
E = 320000
C = 64

NC = 2
NS = 16
NW = NC * NS
EPW = E // NW
CHUNK = 80
NCHUNK = EPW // CHUNK
RPT = N // NS


NBUF = 4
NIB = 6
GRP = 12


def _agg_body(feat, ei, zrows, out, rows, idx, acc, gsems, ssems, isems,
              zsem):
    c = lax.axis_index("c")
    s = lax.axis_index("s")
    zcp = pltpu.async_copy(zrows, acc.at[pl.ds(s * RPT, RPT)], zsem)

    def fetch_idx(t, sl):
        pltpu.async_copy(ei.at[c, s, t], idx.at[sl], isems[sl])

    def wait_idx(t, sl):
        pltpu.make_async_copy(ei.at[c, s, t], idx.at[sl], isems[sl]).wait()

    HC = CHUNK // 2

    def gather(t, sl, b):
        pltpu.async_copy(feat.at[idx.at[sl, 0, pl.ds(0, HC)]],
                         rows.at[b, pl.ds(0, HC)], gsems[b])
        pltpu.async_copy(feat.at[idx.at[sl, 0, pl.ds(HC, HC)]],
                         rows.at[b, pl.ds(HC, HC)], gsems[b])

    def wait_gather(t, sl, b):
        pltpu.make_async_copy(feat.at[idx.at[sl, 0, pl.ds(0, HC)]],
                              rows.at[b, pl.ds(0, HC)], gsems[b]).wait()
        pltpu.make_async_copy(feat.at[idx.at[sl, 0, pl.ds(HC, HC)]],
                              rows.at[b, pl.ds(HC, HC)], gsems[b]).wait()

    def scatter(t, sl, b):
        pltpu.async_copy(rows.at[b], acc.at[idx.at[sl, 1]], ssems[b],
                         add=True)

    def wait_scatter(t, sl, b):
        pltpu.make_async_copy(rows.at[b], acc.at[idx.at[sl, 1]],
                              ssems[b]).wait()

    for t in range(4):
        fetch_idx(t, t)
    wait_idx(0, 0)
    gather(0, 0, 0)
    wait_idx(1, 1)
    gather(1, 1, 1)
    zcp.wait()
    plsc.subcore_barrier()
    for t in (0, 1):
        wait_gather(t, t % 6, t % 4)
        fetch_idx(t + 4, (t + 4) % 6)
        wait_idx(t + 2, (t + 2) % 6)
        gather(t + 2, (t + 2) % 6, (t + 2) % 4)
        scatter(t, t % 6, t % 4)

    def group(i, carry):
        for b in range(GRP):
            t = 2 + i * GRP + b
            sl = (2 + b) % 6
            rb = (2 + b) % 4
            wait_gather(t, sl, rb)
            wait_scatter(t - 2, (sl + 4) % 6, (rb + 2) % 4)
            fetch_idx(t + 4, (sl + 4) % 6)
            wait_idx(t + 2, (sl + 2) % 6)
            gather(t + 2, (sl + 2) % 6, (rb + 2) % 4)
            scatter(t, sl, rb)
        return carry

    lax.fori_loop(0, 9, group, 0)

    for t in range(110, NCHUNK):
        sl = t % 6
        rb = t % 4
        wait_gather(t, sl, rb)
        wait_scatter(t - 2, (t - 2) % 6, (t - 2) % 4)
        if t + 4 < NCHUNK:
            fetch_idx(t + 4, (t + 4) % 6)
        if t + 2 < NCHUNK:
            wait_idx(t + 2, (t + 2) % 6)
            gather(t + 2, (t + 2) % 6, (t + 2) % 4)
        scatter(t, sl, rb)
    wait_scatter(NCHUNK - 2, (NCHUNK - 2) % 6, (NCHUNK - 2) % 4)
    wait_scatter(NCHUNK - 1, (NCHUNK - 1) % 6, (NCHUNK - 1) % 4)

    plsc.subcore_barrier()
    pltpu.sync_copy(acc.at[pl.ds(s * RPT, RPT)], out.at[c, s])


_agg = pl.kernel(
    _agg_body,
    out_type=jax.ShapeDtypeStruct((NC, NS, RPT, D), jnp.float32),
    mesh=plsc.VectorSubcoreMesh(core_axis_name="c", subcore_axis_name="s"),
    scratch_types=[
        pltpu.VMEM((NBUF, CHUNK, D), jnp.float32),
        pltpu.VMEM((NIB, 2, CHUNK), jnp.int32),
        pltpu.VMEM_SHARED((N, D), jnp.float32),
        [pltpu.SemaphoreType.DMA] * NBUF,
        [pltpu.SemaphoreType.DMA] * NBUF,
        [pltpu.SemaphoreType.DMA] * NIB,
        pltpu.SemaphoreType.DMA,
    ],
)


RB = 2000


def _mlp1_body(x_ref, p_ref, wa_ref, ba_ref, wb_ref, bb_ref, o_ref):
    h = x_ref[...] + p_ref[0] + p_ref[1]
    a = jnp.dot(h, wa_ref[...], preferred_element_type=jnp.float32)
    a = jnp.maximum(a + ba_ref[...], 0.0)
    b = jnp.dot(a, wb_ref[...], preferred_element_type=jnp.float32)
    o_ref[...] = jnp.maximum(b + bb_ref[...], 0.0)


def _mlp2_body(h_ref, p_ref, wa_ref, ba_ref, wb_ref, bb_ref, wl_ref,
               bl_ref, o_ref):
    h = h_ref[...] + p_ref[0] + p_ref[1]
    a = jnp.dot(h, wa_ref[...], preferred_element_type=jnp.float32)
    a = jnp.maximum(a + ba_ref[...], 0.0)
    g = jnp.dot(a, wb_ref[...], preferred_element_type=jnp.float32)
    g = jnp.maximum(g + bb_ref[...], 0.0)
    l = jnp.dot(g, wl_ref[...], preferred_element_type=jnp.float32)
    l = l + bl_ref[...]
    m = jnp.max(l, axis=1, keepdims=True)
    e = jnp.exp(l - m)
    lse = jnp.log(jnp.sum(e, axis=1, keepdims=True))
    o_ref[...] = l - m - lse


def _rep(i):
    return (0, 0)


def _mlp1(x, p, Wa, ba, Wb, bb):
    return pl.pallas_call(
        _mlp1_body,
        grid=(N // RB,),
        in_specs=[
            pl.BlockSpec((RB, D), lambda i: (i, 0)),
            pl.BlockSpec((NC, RB, D), lambda i: (0, i, 0)),
            pl.BlockSpec((D, H), _rep),
            pl.BlockSpec((1, H), _rep),
            pl.BlockSpec((H, H), _rep),
            pl.BlockSpec((1, H), _rep),
        ],
        out_specs=pl.BlockSpec((RB, H), lambda i: (i, 0)),
        out_shape=jax.ShapeDtypeStruct((N, H), jnp.float32),
    )(x, p, Wa, ba, Wb, bb)


def _mlp2(h, p, Wa, ba, Wb, bb, Wl, bl):
    return pl.pallas_call(
        _mlp2_body,
        grid=(N // RB,),
        in_specs=[
            pl.BlockSpec((RB, H), lambda i: (i, 0)),
            pl.BlockSpec((NC, RB, H), lambda i: (0, i, 0)),
            pl.BlockSpec((H, H), _rep),
            pl.BlockSpec((1, H), _rep),
            pl.BlockSpec((H, H), _rep),
            pl.BlockSpec((1, H), _rep),
            pl.BlockSpec((H, C), _rep),
            pl.BlockSpec((1, C), _rep),
        ],
        out_specs=pl.BlockSpec((RB, C), lambda i: (i, 0)),
        out_shape=jax.ShapeDtypeStruct((N, C), jnp.float32),
    )(h, p, Wa, ba, Wb, bb, Wl, bl)


def kernel(x, edge_index, W1a, b1a, W1b, b1b, g1, be1,
           W2a, b2a, W2b, b2b, g2, be2, Wl, bl):
    s = 1.0 / jnp.sqrt(jnp.float32(1.0 + 1e-5))
    sc1 = g1 * s
    W1bf = W1b * sc1[None, :]
    b1bf = b1b * sc1 + be1
    sc2 = g2 * s
    W2bf = W2b * sc2[None, :]
    b2bf = b2b * sc2 + be2

    ei = edge_index.reshape(2, NC, NS, NCHUNK, CHUNK).transpose(1, 2, 3, 0, 4)
    zrows = jnp.zeros((RPT, D), jnp.float32)

    p1 = _agg(x, ei, zrows).reshape(NC, N, D)
    h1 = _mlp1(x, p1, W1a, b1a.reshape(1, H), W1bf, b1bf.reshape(1, H))
    p2 = _agg(h1, ei, zrows).reshape(NC, N, H)
    return _mlp2(h1, p2, W2a, b2a.reshape(1, H), W2bf, b2bf.reshape(1, H),
                 Wl, bl.reshape(1, C))


# speedup vs baseline: 1.0030x; 1.0030x over previous
import jax
import jax.numpy as jnp
from jax import lax
from jax.experimental import pallas as pl
from jax.experimental.pallas import tpu as pltpu
from jax.experimental.pallas import tpu_sc as plsc

N = 10000
D = 128
H = 128
E = 320000
C = 64

NC = 2
NS = 16
NW = NC * NS
EPW = E // NW
CHUNK = 80
NCHUNK = EPW // CHUNK
RPT = N // NS


NBUF = 4
NIB = 6
GRP = 12


def _agg_body(feat, ei, zrows, out, rows, idx, acc, gsems, ssems, isems,
              zsem):
    c = lax.axis_index("c")
    s = lax.axis_index("s")
    zcp = pltpu.async_copy(zrows, acc.at[pl.ds(s * RPT, RPT)], zsem)

    def fetch_idx(t, sl):
        pltpu.async_copy(ei.at[c, s, t], idx.at[sl], isems[sl])

    def wait_idx(t, sl):
        pltpu.make_async_copy(ei.at[c, s, t], idx.at[sl], isems[sl]).wait()

    def gather(t, sl, b):
        pltpu.async_copy(feat.at[idx.at[sl, 0]], rows.at[b], gsems[b])

    def wait_gather(t, sl, b):
        pltpu.make_async_copy(feat.at[idx.at[sl, 0]], rows.at[b],
                              gsems[b]).wait()

    def scatter(t, sl, b):
        pltpu.async_copy(rows.at[b], acc.at[idx.at[sl, 1]], ssems[b],
                         add=True)

    def wait_scatter(t, sl, b):
        pltpu.make_async_copy(rows.at[b], acc.at[idx.at[sl, 1]],
                              ssems[b]).wait()

    for t in range(4):
        fetch_idx(t, t)
    wait_idx(0, 0)
    gather(0, 0, 0)
    wait_idx(1, 1)
    gather(1, 1, 1)
    zcp.wait()
    plsc.subcore_barrier()
    for t in (0, 1):
        wait_gather(t, t % 6, t % 4)
        fetch_idx(t + 4, (t + 4) % 6)
        wait_idx(t + 2, (t + 2) % 6)
        gather(t + 2, (t + 2) % 6, (t + 2) % 4)
        scatter(t, t % 6, t % 4)

    def group(i, carry):
        for b in range(GRP):
            t = 2 + i * GRP + b
            sl = (2 + b) % 6
            rb = (2 + b) % 4
            wait_gather(t, sl, rb)
            wait_scatter(t - 2, (sl + 4) % 6, (rb + 2) % 4)
            fetch_idx(t + 4, (sl + 4) % 6)
            wait_idx(t + 2, (sl + 2) % 6)
            gather(t + 2, (sl + 2) % 6, (rb + 2) % 4)
            scatter(t, sl, rb)
        return carry

    lax.fori_loop(0, 9, group, 0)

    for t in range(110, NCHUNK):
        sl = t % 6
        rb = t % 4
        wait_gather(t, sl, rb)
        wait_scatter(t - 2, (t - 2) % 6, (t - 2) % 4)
        if t + 4 < NCHUNK:
            fetch_idx(t + 4, (t + 4) % 6)
        if t + 2 < NCHUNK:
            wait_idx(t + 2, (t + 2) % 6)
            gather(t + 2, (t + 2) % 6, (t + 2) % 4)
        scatter(t, sl, rb)
    wait_scatter(NCHUNK - 2, (NCHUNK - 2) % 6, (NCHUNK - 2) % 4)
    wait_scatter(NCHUNK - 1, (NCHUNK - 1) % 6, (NCHUNK - 1) % 4)

    plsc.subcore_barrier()
    pltpu.sync_copy(acc.at[pl.ds(s * RPT, RPT)], out.at[c, s])


_agg = pl.kernel(
    _agg_body,
    out_type=jax.ShapeDtypeStruct((NC, NS, RPT, D), jnp.float32),
    mesh=plsc.VectorSubcoreMesh(core_axis_name="c", subcore_axis_name="s"),
    scratch_types=[
        pltpu.VMEM((NBUF, CHUNK, D), jnp.float32),
        pltpu.VMEM((NIB, 2, CHUNK), jnp.int32),
        pltpu.VMEM_SHARED((N, D), jnp.float32),
        [pltpu.SemaphoreType.DMA] * NBUF,
        [pltpu.SemaphoreType.DMA] * NBUF,
        [pltpu.SemaphoreType.DMA] * NIB,
        pltpu.SemaphoreType.DMA,
    ],
)


RB = 2000


def _mlp1_body(x_ref, p_ref, wa_ref, ba_ref, wb_ref, bb_ref, o_ref):
    h = x_ref[...] + p_ref[0] + p_ref[1]
    a = jnp.dot(h, wa_ref[...], preferred_element_type=jnp.float32)
    a = jnp.maximum(a + ba_ref[...], 0.0)
    b = jnp.dot(a, wb_ref[...], preferred_element_type=jnp.float32)
    o_ref[...] = jnp.maximum(b + bb_ref[...], 0.0)


def _mlp2_body(h_ref, p_ref, wa_ref, ba_ref, wb_ref, bb_ref, wl_ref,
               bl_ref, o_ref):
    h = h_ref[...] + p_ref[0] + p_ref[1]
    a = jnp.dot(h, wa_ref[...], preferred_element_type=jnp.float32)
    a = jnp.maximum(a + ba_ref[...], 0.0)
    g = jnp.dot(a, wb_ref[...], preferred_element_type=jnp.float32)
    g = jnp.maximum(g + bb_ref[...], 0.0)
    l = jnp.dot(g, wl_ref[...], preferred_element_type=jnp.float32)
    l = l + bl_ref[...]
    m = jnp.max(l, axis=1, keepdims=True)
    e = jnp.exp(l - m)
    lse = jnp.log(jnp.sum(e, axis=1, keepdims=True))
    o_ref[...] = l - m - lse


def _rep(i):
    return (0, 0)


def _mlp1(x, p, Wa, ba, Wb, bb):
    return pl.pallas_call(
        _mlp1_body,
        grid=(N // RB,),
        in_specs=[
            pl.BlockSpec((RB, D), lambda i: (i, 0)),
            pl.BlockSpec((NC, RB, D), lambda i: (0, i, 0)),
            pl.BlockSpec((D, H), _rep),
            pl.BlockSpec((1, H), _rep),
            pl.BlockSpec((H, H), _rep),
            pl.BlockSpec((1, H), _rep),
        ],
        out_specs=pl.BlockSpec((RB, H), lambda i: (i, 0)),
        out_shape=jax.ShapeDtypeStruct((N, H), jnp.float32),
    )(x, p, Wa, ba, Wb, bb)


def _mlp2(h, p, Wa, ba, Wb, bb, Wl, bl):
    return pl.pallas_call(
        _mlp2_body,
        grid=(N // RB,),
        in_specs=[
            pl.BlockSpec((RB, H), lambda i: (i, 0)),
            pl.BlockSpec((NC, RB, H), lambda i: (0, i, 0)),
            pl.BlockSpec((H, H), _rep),
            pl.BlockSpec((1, H), _rep),
            pl.BlockSpec((H, H), _rep),
            pl.BlockSpec((1, H), _rep),
            pl.BlockSpec((H, C), _rep),
            pl.BlockSpec((1, C), _rep),
        ],
        out_specs=pl.BlockSpec((RB, C), lambda i: (i, 0)),
        out_shape=jax.ShapeDtypeStruct((N, C), jnp.float32),
    )(h, p, Wa, ba, Wb, bb, Wl, bl)


def kernel(x, edge_index, W1a, b1a, W1b, b1b, g1, be1,
           W2a, b2a, W2b, b2b, g2, be2, Wl, bl):
    s = 1.0 / jnp.sqrt(jnp.float32(1.0 + 1e-5))
    sc1 = g1 * s
    W1bf = W1b * sc1[None, :]
    b1bf = b1b * sc1 + be1
    sc2 = g2 * s
    W2bf = W2b * sc2[None, :]
    b2bf = b2b * sc2 + be2

    ei = edge_index.reshape(2, NC, NS, NCHUNK, CHUNK).transpose(1, 2, 3, 0, 4)
    zrows = jnp.zeros((RPT, D), jnp.float32)

    p1 = _agg(x, ei, zrows).reshape(NC, N, D)
    h1 = _mlp1(x, p1, W1a, b1a.reshape(1, H), W1bf, b1bf.reshape(1, H))
    p2 = _agg(h1, ei, zrows).reshape(NC, N, H)
    return _mlp2(h1, p2, W2a, b2a.reshape(1, H), W2bf, b2bf.reshape(1, H),
                 Wl, bl.reshape(1, C))
